# baseline (speedup 1.0000x reference)
import jax
import jax.numpy as jnp
from jax import lax
from jax.experimental import pallas as pl
from jax.experimental.pallas import tpu as pltpu
from jax.experimental.pallas import tpu_sc as plsc

VOCAB_ROWS = 1000000
EMB_DIM = 64
N_SEQ = 4096
SEQ_LEN = 200
N_TOK = N_SEQ * SEQ_LEN

_info = plsc.get_sparse_core_info()
_NC = _info.num_cores
_NS = _info.num_subcores
_NL = _info.num_lanes
_NW = _NC * _NS
_TOK_PER_W = N_TOK // _NW
_CHUNK = 256
_NB = 4
_N_CHUNKS = _TOK_PER_W // _CHUNK


_TILE = 512
_N_FULL = VOCAB_ROWS // _TILE
_TAIL = VOCAB_ROWS - _N_FULL * _TILE
_PAIR_ROWS = VOCAB_ROWS // 2


def _format_body(tt_hbm, tail_hbm, out_hbm, stage, outbuf, tailbuf, s0, s1):
    wid = lax.axis_index("s") * _NC + lax.axis_index("c")
    lane = lax.iota(jnp.int32, _NL)
    ssems = (s0, s1)
    n_i = (_N_FULL - 1 - wid) // _NW + 1

    def start_stage(i, b):
        t = wid + i * _NW
        pltpu.async_copy(
            tt_hbm.at[:, pl.ds(t * _TILE, _TILE)], stage.at[b], ssems[b])

    def wait_stage(b):
        pltpu.make_async_copy(
            tt_hbm.at[:, pl.ds(0, _TILE)], stage.at[b], ssems[b]).wait()

    def transpose(b, n_tok):
        def jgroup(g, carry):
            j = g * _NL + lane
            prow = lax.shift_right_logical(j, 1)
            pcol0 = lax.shift_left(lax.bitwise_and(j, 1), 6)
            for d in range(EMB_DIM):
                v = stage[b, d, pl.ds(g * _NL, _NL)]
                plsc.store_scatter(outbuf, [prow, pcol0 + d], v)
            return carry
        lax.fori_loop(0, n_tok // _NL, jgroup, 0)

    start_stage(0, 0)

    def block_pair(k, carry):
        for b in range(2):
            i = k * 2 + b

            @pl.when(i < n_i)
            def _():
                t = wid + i * _NW
                wait_stage(b)

                @pl.when(i + 1 < n_i)
                def _():
                    start_stage(i + 1, 1 - b)

                transpose(b, _TILE)
                pltpu.sync_copy(
                    outbuf,
                    out_hbm.at[pl.ds(t * (_TILE // 2), _TILE // 2), :])
        return carry

    lax.fori_loop(0, (n_i + 1) // 2, block_pair, 0)

    @pl.when(wid == 0)
    def _():
        pltpu.sync_copy(tail_hbm, tailbuf)
        pltpu.sync_copy(
            tailbuf, out_hbm.at[pl.ds(_N_FULL * _TILE // 2, _TAIL // 2), :])


def _emb_body(ids_hbm, table_hbm, out_hbm, idx_v, rows_v,
              g0, g1, g2, g3, w0, w1, w2, w3):
    gsems = (g0, g1, g2, g3)
    wsems = (w0, w1, w2, w3)
    wid = lax.axis_index("s") * _NC + lax.axis_index("c")
    base = wid * _TOK_PER_W
    pltpu.sync_copy(ids_hbm.at[pl.ds(base, _TOK_PER_W)], idx_v)

    def start_gather(c, b):
        pltpu.async_copy(
            table_hbm.at[idx_v.at[pl.ds(c * _CHUNK, _CHUNK)]],
            rows_v.at[b], gsems[b])

    def wait_gather(b):
        pltpu.make_async_copy(
            table_hbm.at[idx_v.at[pl.ds(0, _CHUNK)]],
            rows_v.at[b], gsems[b]).wait()

    def start_write(c, b):
        pltpu.async_copy(
            rows_v.at[b],
            out_hbm.at[pl.ds(base + c * _CHUNK, _CHUNK), pl.ds(0, EMB_DIM)],
            wsems[b])

    def wait_write(b):
        pltpu.make_async_copy(
            rows_v.at[b],
            out_hbm.at[pl.ds(base, _CHUNK), pl.ds(0, EMB_DIM)],
            wsems[b]).wait()

    for b in range(_NB):
        start_gather(b, b)

    def group(k, carry):
        for b in range(_NB):
            c = k * _NB + b
            wait_gather(b)
            start_write(c, b)
            wait_write(b)
            start_gather(c + _NB, b)
        return carry

    lax.fori_loop(0, _N_CHUNKS // _NB - 1, group, 0)

    for b in range(_NB):
        c = (_N_CHUNKS // _NB - 1) * _NB + b
        wait_gather(b)
        start_write(c, b)
        wait_write(b)


@jax.jit
def kernel(input_ids, table):
    fmt = pl.kernel(
        _format_body,
        mesh=plsc.VectorSubcoreMesh(core_axis_name="c", subcore_axis_name="s"),
        out_type=jax.ShapeDtypeStruct((_PAIR_ROWS, 2 * EMB_DIM), jnp.float32),
        scratch_types=[
            pltpu.VMEM((2, EMB_DIM, _TILE), jnp.float32),
            pltpu.VMEM((_TILE // 2, 2 * EMB_DIM), jnp.float32),
            pltpu.VMEM((_TAIL // 2, 2 * EMB_DIM), jnp.float32),
            pltpu.SemaphoreType.DMA,
            pltpu.SemaphoreType.DMA,
        ],
        compiler_params=pltpu.CompilerParams(needs_layout_passes=False),
    )
    t_tail = jnp.reshape(table[_N_FULL * _TILE:, :], (_TAIL // 2, 2 * EMB_DIM))
    t2 = fmt(jnp.transpose(table), t_tail)
    t_lin = jnp.reshape(t2, (VOCAB_ROWS, EMB_DIM))
    ids_flat = jnp.reshape(input_ids.astype(jnp.int32), (N_TOK,))
    gather = pl.kernel(
        _emb_body,
        mesh=plsc.VectorSubcoreMesh(core_axis_name="c", subcore_axis_name="s"),
        out_type=jax.ShapeDtypeStruct((N_TOK, 2 * EMB_DIM), jnp.float32),
        scratch_types=[
            pltpu.VMEM((_TOK_PER_W,), jnp.int32),
            pltpu.VMEM((_NB, _CHUNK, EMB_DIM), jnp.float32),
        ] + [pltpu.SemaphoreType.DMA] * (2 * _NB),
        compiler_params=pltpu.CompilerParams(use_tc_tiling_on_sc=False),
    )
    out128 = gather(ids_flat, t_lin)
    out3 = jnp.reshape(out128, (N_SEQ, SEQ_LEN, 2 * EMB_DIM))
    return out3[:, :, :EMB_DIM]


# speedup vs baseline: 1.7588x; 1.7588x over previous
import jax
import jax.numpy as jnp
from jax import lax
from jax.experimental import pallas as pl
from jax.experimental.pallas import tpu as pltpu
from jax.experimental.pallas import tpu_sc as plsc

VOCAB_ROWS = 1000000
EMB_DIM = 64
N_SEQ = 4096
SEQ_LEN = 200
N_TOK = N_SEQ * SEQ_LEN

_info = plsc.get_sparse_core_info()
_NC = _info.num_cores
_NS = _info.num_subcores
_NW = _NC * _NS
_TOK_PER_W = N_TOK // _NW
_CHUNK = 256
_NB = 5
_N_CHUNKS = _TOK_PER_W // _CHUNK


def _emb_body(ids_hbm, table_hbm, out_hbm, idx_v, rows_v,
              g0, g1, g2, g3, g4, w0, w1, w2, w3, w4):
    gsems = (g0, g1, g2, g3, g4)
    wsems = (w0, w1, w2, w3, w4)
    wid = lax.axis_index("s") * _NC + lax.axis_index("c")
    base = wid * _TOK_PER_W
    pltpu.sync_copy(ids_hbm.at[pl.ds(base, _TOK_PER_W)], idx_v)

    def start_gather(c, b):
        pltpu.async_copy(
            table_hbm.at[idx_v.at[pl.ds(c * _CHUNK, _CHUNK)]],
            rows_v.at[b], gsems[b])

    def wait_gather(b):
        pltpu.make_async_copy(
            table_hbm.at[idx_v.at[pl.ds(0, _CHUNK)]],
            rows_v.at[b], gsems[b]).wait()

    def start_write(c, b):
        pltpu.async_copy(
            rows_v.at[b],
            out_hbm.at[pl.ds(base + c * _CHUNK, _CHUNK), pl.ds(0, EMB_DIM)],
            wsems[b])

    def wait_write(b):
        pltpu.make_async_copy(
            rows_v.at[b],
            out_hbm.at[pl.ds(base, _CHUNK), pl.ds(0, EMB_DIM)],
            wsems[b]).wait()

    for b in range(_NB):
        start_gather(b, b)

    def group(k, carry):
        for b in range(_NB):
            c = k * _NB + b
            wait_gather(b)
            start_write(c, b)
            wait_write(b)
            start_gather(c + _NB, b)
        return carry

    lax.fori_loop(0, _N_CHUNKS // _NB - 1, group, 0)

    for b in range(_NB):
        c = (_N_CHUNKS // _NB - 1) * _NB + b
        wait_gather(b)
        start_write(c, b)
        wait_write(b)


@jax.jit
def kernel(input_ids, table):
    ids_flat = jnp.reshape(input_ids.astype(jnp.int32), (N_TOK,))
    gather = pl.kernel(
        _emb_body,
        mesh=plsc.VectorSubcoreMesh(core_axis_name="c", subcore_axis_name="s"),
        out_type=jax.ShapeDtypeStruct((N_TOK, 2 * EMB_DIM), jnp.float32),
        scratch_types=[
            pltpu.VMEM((_TOK_PER_W,), jnp.int32),
            pltpu.VMEM((_NB, _CHUNK, EMB_DIM), jnp.float32),
        ] + [pltpu.SemaphoreType.DMA] * (2 * _NB),
        compiler_params=pltpu.CompilerParams(use_tc_tiling_on_sc=False),
    )
    out128 = gather(ids_flat, table)
    out3 = jnp.reshape(out128, (N_SEQ, SEQ_LEN, 2 * EMB_DIM))
    return out3[:, :, :EMB_DIM]
